# baseline (speedup 1.0000x reference)
import functools

import jax
import jax.numpy as jnp
from jax import lax
from jax.experimental import pallas as pl
from jax.experimental.pallas import tpu as pltpu
from jax.experimental.pallas import tpu_sc as plsc

N = 10000
E = 160000
DIN = 256
DH = 512
G = 8

NC = 2
NS = 16
TRB = 128
EPAD = 163840
TPT = EPAD // NS
NT = TPT // TRB
NPAD = 10112
SL = NPAD // NS

R = 1000
NB = N // R


NBUF = 2
IDXR = 40
NSTG = NT // IDXR


def _make_agg(C):
    CPS = C // NC
    mesh = plsc.VectorSubcoreMesh(core_axis_name="c", subcore_axis_name="s")

    @functools.partial(
        pl.kernel,
        mesh=mesh,
        out_type=jax.ShapeDtypeStruct((C * NPAD, 128), jnp.float32),
        scratch_types=[
            pltpu.VMEM((IDXR, TRB), jnp.int32),
            pltpu.VMEM((IDXR, TRB), jnp.int32),
            pltpu.VMEM((NBUF, TRB, 128), jnp.float32),
            pltpu.VMEM_SHARED((NPAD, 128), jnp.float32),
            pltpu.SemaphoreType.DMA,
            pltpu.SemaphoreType.DMA,
            pltpu.SemaphoreType.DMA,
            pltpu.SemaphoreType.DMA,
        ],
    )
    def k(table, src2, dst2, zeros, out, src_v, dst_v, rows, acc,
          sg0, sg1, ss0, ss1):
        sgs = [sg0, sg1]
        sss = [ss0, ss1]
        cid = lax.axis_index("c")
        sid = lax.axis_index("s")
        for j in range(CPS):
            chunk = cid * CPS + j
            pltpu.sync_copy(zeros.at[pl.ds(sid * SL, SL)],
                            acc.at[pl.ds(sid * SL, SL)])
            off = chunk * N
            plsc.subcore_barrier()

            for st in range(NSTG):
                base = sid * NT + st * IDXR
                pltpu.sync_copy(src2.at[pl.ds(base, IDXR)], src_v)
                pltpu.sync_copy(dst2.at[pl.ds(base, IDXR)], dst_v)

                def addoff(i, _):
                    for q in range(TRB // 16):
                        src_v[i, pl.ds(q * 16, 16)] = (
                            src_v[i, pl.ds(q * 16, 16)] + off)
                    return 0

                lax.fori_loop(0, IDXR, addoff, 0)

                for b in range(NBUF):
                    pltpu.async_copy(table.at[src_v.at[b]], rows.at[b],
                                     sgs[b])

                def body(i, _):
                    for b in range(NBUF):
                        t = i * NBUF + b
                        pltpu.make_async_copy(table.at[src_v.at[b]],
                                              rows.at[b], sgs[b]).wait()
                        pltpu.async_copy(rows.at[b], acc.at[dst_v.at[t]],
                                         sss[0], add=True)
                        pltpu.make_async_copy(rows.at[b],
                                              acc.at[dst_v.at[t]],
                                              sss[0]).wait()
                        nxt = t + NBUF

                        @pl.when(nxt < IDXR)
                        def _():
                            pltpu.async_copy(table.at[src_v.at[nxt]],
                                             rows.at[b], sgs[b])
                    return 0

                lax.fori_loop(0, IDXR // NBUF, body, 0)
            plsc.subcore_barrier()
            pltpu.sync_copy(acc.at[pl.ds(sid * SL, SL)],
                            out.at[pl.ds(chunk * NPAD + sid * SL, SL)])
            if j + 1 < CPS:
                plsc.subcore_barrier()

    return k


_agg2 = _make_agg(2)
_agg4 = _make_agg(4)


def _make_deg():
    mesh = plsc.VectorSubcoreMesh(core_axis_name="c", subcore_axis_name="s")

    @functools.partial(
        pl.kernel,
        mesh=mesh,
        out_type=jax.ShapeDtypeStruct((NC * NPAD, 128), jnp.float32),
        scratch_types=[
            pltpu.VMEM((NT, TRB), jnp.int32),
            pltpu.VMEM((TRB, 128), jnp.float32),
            pltpu.VMEM_SHARED((NPAD, 128), jnp.float32),
            pltpu.SemaphoreType.DMA,
        ],
    )
    def k(dst2, zeros, ones, out, dst_v, ones_v, acc, ss):
        cid = lax.axis_index("c")
        sid = lax.axis_index("s")
        pltpu.sync_copy(zeros.at[pl.ds(sid * SL, SL)],
                        acc.at[pl.ds(sid * SL, SL)])
        pltpu.sync_copy(dst2.at[pl.ds(cid * (EPAD // TRB) + sid * NT, NT)],
                        dst_v)
        pltpu.sync_copy(ones, ones_v)
        plsc.subcore_barrier()
        K_FIRE = 8

        def body(i, _):
            for b in range(K_FIRE):
                t = i * K_FIRE + b
                pltpu.async_copy(ones_v, acc.at[dst_v.at[t]], ss, add=True)
            for b in range(K_FIRE):
                t = i * K_FIRE + b
                pltpu.make_async_copy(ones_v, acc.at[dst_v.at[t]], ss).wait()
            return 0

        lax.fori_loop(0, NT // K_FIRE, body, 0)
        plsc.subcore_barrier()
        pltpu.sync_copy(acc.at[pl.ds(sid * SL, SL)],
                        out.at[pl.ds(cid * NPAD + sid * SL, SL)])

    return k


_deg_kernel = _make_deg()


def _dense_body(deg_ref, s_ref, h_ref, wlt_ref, wrt_ref, b_ref, o_ref, *,
                relu, cin):
    rdeg = 1.0 / jnp.maximum(deg_ref[...], 1.0)
    acc = jnp.zeros((R, DH), jnp.float32) + b_ref[...]
    for kk in range(cin):
        acc = acc + jnp.dot(s_ref[kk] * rdeg, wlt_ref[kk],
                            preferred_element_type=jnp.float32)
        acc = acc + jnp.dot(h_ref[kk], wrt_ref[kk],
                            preferred_element_type=jnp.float32)
    if relu:
        acc = jnp.maximum(acc, 0.0)
    for kk in range(DH // 128):
        o_ref[kk] = acc[:, kk * 128:(kk + 1) * 128]


def _dense_layer(deg, s3, h3, wlt3, wrt3, b, relu):
    cin = s3.shape[0]
    return pl.pallas_call(
        functools.partial(_dense_body, relu=relu, cin=cin),
        grid=(NB,),
        in_specs=[
            pl.BlockSpec((R, 1), lambda r: (r, 0)),
            pl.BlockSpec((cin, R, 128), lambda r: (0, r, 0)),
            pl.BlockSpec((cin, R, 128), lambda r: (0, r, 0)),
            pl.BlockSpec((cin, 128, DH), lambda r: (0, 0, 0)),
            pl.BlockSpec((cin, 128, DH), lambda r: (0, 0, 0)),
            pl.BlockSpec((1, DH), lambda r: (0, 0)),
        ],
        out_specs=pl.BlockSpec((DH // 128, R, 128), lambda r: (0, r, 0)),
        out_shape=jax.ShapeDtypeStruct((DH // 128, N, 128), jnp.float32),
    )(deg, s3, h3, wlt3, wrt3, b)


def _pool_body(fw_ref, bw_ref, batch_ref, xo_ref, ge_ref):
    r = pl.program_id(0)
    parts = [fw_ref[kk] for kk in range(DH // 128)]
    parts += [bw_ref[kk] for kk in range(DH // 128)]
    xcat = jnp.concatenate(parts, axis=1)
    xo_ref[...] = xcat
    batch = batch_ref[...]

    @pl.when(r == 0)
    def _init():
        ge_ref[...] = jnp.full((G, 2 * DH), -jnp.inf, jnp.float32)

    neg = jnp.float32(-jnp.inf)
    rows = []
    for g in range(G):
        rows.append(jnp.max(jnp.where(batch == g, xcat, neg), axis=0))
    ge_ref[...] = jnp.maximum(ge_ref[...], jnp.stack(rows, axis=0))


def _pool(fw3, bw3, batch):
    return pl.pallas_call(
        _pool_body,
        grid=(NB,),
        in_specs=[
            pl.BlockSpec((DH // 128, R, 128), lambda r: (0, r, 0)),
            pl.BlockSpec((DH // 128, R, 128), lambda r: (0, r, 0)),
            pl.BlockSpec((R, 1), lambda r: (r, 0)),
        ],
        out_specs=[
            pl.BlockSpec((R, 2 * DH), lambda r: (r, 0)),
            pl.BlockSpec((G, 2 * DH), lambda r: (0, 0)),
        ],
        out_shape=[
            jax.ShapeDtypeStruct((N, 2 * DH), jnp.float32),
            jax.ShapeDtypeStruct((G, 2 * DH), jnp.float32),
        ],
    )(fw3, bw3, batch)


def _sage(x3, srcp, dstp, deg, params, zeros128):
    h3 = x3
    n_layers = len(params)
    for i, (wl, wr, b) in enumerate(params):
        cin = h3.shape[0]
        table = h3.reshape(cin * N, 128)
        agg = _agg2 if cin == 2 else _agg4
        s3 = agg(table, srcp, dstp, zeros128).reshape(cin, NPAD, 128)
        wlt3 = wl.T.reshape(cin, 128, DH)
        wrt3 = wr.T.reshape(cin, 128, DH)
        h3 = _dense_layer(deg, s3, h3, wlt3, wrt3, b[None, :],
                          relu=(i < n_layers - 1))
    return h3


def kernel(x, api_feat, root_feat, root_id, fw_edge_idx, bw_edge_idx, batch,
           fw_params, bw_params):
    pad = EPAD - E
    fw_dst, fw_src = lax.sort((fw_edge_idx[1].astype(jnp.int32),
                               fw_edge_idx[0].astype(jnp.int32)),
                              num_keys=1)
    bw_dst, bw_src = lax.sort((bw_edge_idx[1].astype(jnp.int32),
                               bw_edge_idx[0].astype(jnp.int32)),
                              num_keys=1)
    zpad = jnp.zeros((pad,), jnp.int32)
    npad = jnp.full((pad,), N, jnp.int32)
    fw_src_p = jnp.concatenate([fw_src, zpad]).reshape(EPAD // TRB, TRB)
    fw_dst_p = jnp.concatenate([fw_dst, npad]).reshape(EPAD // TRB, TRB)
    bw_src_p = jnp.concatenate([bw_src, zpad]).reshape(EPAD // TRB, TRB)
    bw_dst_p = jnp.concatenate([bw_dst, npad]).reshape(EPAD // TRB, TRB)

    zeros128 = jnp.zeros((NPAD, 128), jnp.float32)
    ones128 = jnp.ones((TRB, 128), jnp.float32)

    dst2 = jnp.concatenate([fw_dst_p, bw_dst_p], axis=0)
    degs = _deg_kernel(dst2, zeros128, ones128).reshape(NC, NPAD, 128)
    deg_fw = degs[0, :N, 0:1]
    deg_bw = degs[1, :N, 0:1]

    x3 = x.reshape(N, DIN // 128, 128).transpose(1, 0, 2)

    fw3 = _sage(x3, fw_src_p, fw_dst_p, deg_fw, fw_params, zeros128)
    bw3 = _sage(x3, bw_src_p, bw_dst_p, deg_bw, bw_params, zeros128)

    x_out, graph_embed = _pool(fw3, bw3, batch.astype(jnp.int32)[:, None])
    return (x_out, graph_embed)


# speedup vs baseline: 1.0982x; 1.0982x over previous
import functools

import jax
import jax.numpy as jnp
from jax import lax
from jax.experimental import pallas as pl
from jax.experimental.pallas import tpu as pltpu
from jax.experimental.pallas import tpu_sc as plsc

N = 10000
E = 160000
DIN = 256
DH = 512
G = 8

NC = 2
NS = 16
TRB = 128
EPAD = 163840
TPT = EPAD // NS
NT = TPT // TRB
NPAD = 10112
SL = NPAD // NS

R = 1000
NB = N // R

NBUF = 2
IDXR = 40
NSTG = NT // IDXR


def _make_agg(C):
    CPS = C // NC
    mesh = plsc.VectorSubcoreMesh(core_axis_name="c", subcore_axis_name="s")

    @functools.partial(
        pl.kernel,
        mesh=mesh,
        out_type=jax.ShapeDtypeStruct((C * NPAD, 128), jnp.float32),
        scratch_types=[
            pltpu.VMEM((IDXR, TRB), jnp.int32),
            pltpu.VMEM((IDXR, TRB), jnp.int32),
            pltpu.VMEM((NBUF, TRB, 128), jnp.float32),
            pltpu.VMEM_SHARED((NPAD, 128), jnp.float32),
            pltpu.SemaphoreType.DMA,
            pltpu.SemaphoreType.DMA,
            pltpu.SemaphoreType.DMA,
        ],
    )
    def k(table, src2, dst2, zeros, out, src_v, dst_v, rows, acc,
          sg0, sg1, ss):
        sgs = [sg0, sg1]
        cid = lax.axis_index("c")
        sid = lax.axis_index("s")
        for j in range(CPS):
            chunk = cid * CPS + j
            pltpu.sync_copy(zeros.at[pl.ds(sid * SL, SL)],
                            acc.at[pl.ds(sid * SL, SL)])
            off = chunk * N
            plsc.subcore_barrier()

            for st in range(NSTG):
                base = sid * NT + st * IDXR
                pltpu.sync_copy(src2.at[pl.ds(base, IDXR)], src_v)
                pltpu.sync_copy(dst2.at[pl.ds(base, IDXR)], dst_v)

                def addoff(i, _):
                    for q in range(TRB // 16):
                        src_v[i, pl.ds(q * 16, 16)] = (
                            src_v[i, pl.ds(q * 16, 16)] + off)
                    return 0

                lax.fori_loop(0, IDXR, addoff, 0)

                for b in range(NBUF):
                    pltpu.async_copy(table.at[src_v.at[b]], rows.at[b],
                                     sgs[b])

                def body(i, _):
                    for b in range(NBUF):
                        t = i * NBUF + b
                        pltpu.make_async_copy(table.at[src_v.at[b]],
                                              rows.at[b], sgs[b]).wait()
                        pltpu.async_copy(rows.at[b], acc.at[dst_v.at[t]],
                                         ss, add=True)
                        pltpu.make_async_copy(rows.at[b],
                                              acc.at[dst_v.at[t]],
                                              ss).wait()
                        nxt = t + NBUF

                        @pl.when(nxt < IDXR)
                        def _():
                            pltpu.async_copy(table.at[src_v.at[nxt]],
                                             rows.at[b], sgs[b])
                    return 0

                lax.fori_loop(0, IDXR // NBUF, body, 0)
            plsc.subcore_barrier()
            pltpu.sync_copy(acc.at[pl.ds(sid * SL, SL)],
                            out.at[pl.ds(chunk * NPAD + sid * SL, SL)])
            if j + 1 < CPS:
                plsc.subcore_barrier()

    return k


_agg2 = _make_agg(2)
_agg4 = _make_agg(4)


def _make_deg(W=128):
    mesh = plsc.VectorSubcoreMesh(core_axis_name="c", subcore_axis_name="s")

    @functools.partial(
        pl.kernel,
        mesh=mesh,
        out_type=jax.ShapeDtypeStruct((NC * NPAD, W), jnp.float32),
        scratch_types=[
            pltpu.VMEM((NT, TRB), jnp.int32),
            pltpu.VMEM((TRB, W), jnp.float32),
            pltpu.VMEM_SHARED((NPAD, W), jnp.float32),
            pltpu.SemaphoreType.DMA,
        ],
    )
    def k(dst2, zeros, ones, out, dst_v, ones_v, acc, ss):
        cid = lax.axis_index("c")
        sid = lax.axis_index("s")
        pltpu.sync_copy(zeros.at[pl.ds(sid * SL, SL)],
                        acc.at[pl.ds(sid * SL, SL)])
        pltpu.sync_copy(dst2.at[pl.ds(cid * (EPAD // TRB) + sid * NT, NT)],
                        dst_v)
        pltpu.sync_copy(ones, ones_v)
        plsc.subcore_barrier()
        K_FIRE = 8

        def body(i, _):
            for b in range(K_FIRE):
                t = i * K_FIRE + b
                pltpu.async_copy(ones_v, acc.at[dst_v.at[t]], ss, add=True)
            for b in range(K_FIRE):
                t = i * K_FIRE + b
                pltpu.make_async_copy(ones_v, acc.at[dst_v.at[t]], ss).wait()
            return 0

        lax.fori_loop(0, NT // K_FIRE, body, 0)
        plsc.subcore_barrier()
        pltpu.sync_copy(acc.at[pl.ds(sid * SL, SL)],
                        out.at[pl.ds(cid * NPAD + sid * SL, SL)])

    return k


_deg_kernel = _make_deg()


def _dense_body(deg_ref, s_ref, h_ref, wlt_ref, wrt_ref, b_ref, o_ref, *,
                relu, cin):
    rdeg = 1.0 / jnp.maximum(deg_ref[...], 1.0)
    acc = jnp.zeros((R, DH), jnp.float32) + b_ref[...]
    for kk in range(cin):
        acc = acc + jnp.dot(s_ref[kk] * rdeg, wlt_ref[kk],
                            preferred_element_type=jnp.float32)
        acc = acc + jnp.dot(h_ref[kk], wrt_ref[kk],
                            preferred_element_type=jnp.float32)
    if relu:
        acc = jnp.maximum(acc, 0.0)
    for kk in range(DH // 128):
        o_ref[kk] = acc[:, kk * 128:(kk + 1) * 128]


def _dense_layer(deg, s3, h3, wlt3, wrt3, b, relu):
    cin = s3.shape[0]
    return pl.pallas_call(
        functools.partial(_dense_body, relu=relu, cin=cin),
        grid=(NB,),
        in_specs=[
            pl.BlockSpec((R, 1), lambda r: (r, 0)),
            pl.BlockSpec((cin, R, 128), lambda r: (0, r, 0)),
            pl.BlockSpec((cin, R, 128), lambda r: (0, r, 0)),
            pl.BlockSpec((cin, 128, DH), lambda r: (0, 0, 0)),
            pl.BlockSpec((cin, 128, DH), lambda r: (0, 0, 0)),
            pl.BlockSpec((1, DH), lambda r: (0, 0)),
        ],
        out_specs=pl.BlockSpec((DH // 128, R, 128), lambda r: (0, r, 0)),
        out_shape=jax.ShapeDtypeStruct((DH // 128, N, 128), jnp.float32),
    )(deg, s3, h3, wlt3, wrt3, b)


def _pool_body(fw_ref, bw_ref, batch_ref, xo_ref, ge_ref):
    r = pl.program_id(0)
    parts = [fw_ref[kk] for kk in range(DH // 128)]
    parts += [bw_ref[kk] for kk in range(DH // 128)]
    xcat = jnp.concatenate(parts, axis=1)
    xo_ref[...] = xcat
    batch = batch_ref[...]

    @pl.when(r == 0)
    def _init():
        ge_ref[...] = jnp.full((G, 2 * DH), -jnp.inf, jnp.float32)

    neg = jnp.float32(-jnp.inf)
    rows = []
    for g in range(G):
        rows.append(jnp.max(jnp.where(batch == g, xcat, neg), axis=0))
    ge_ref[...] = jnp.maximum(ge_ref[...], jnp.stack(rows, axis=0))


def _pool(fw3, bw3, batch):
    return pl.pallas_call(
        _pool_body,
        grid=(NB,),
        in_specs=[
            pl.BlockSpec((DH // 128, R, 128), lambda r: (0, r, 0)),
            pl.BlockSpec((DH // 128, R, 128), lambda r: (0, r, 0)),
            pl.BlockSpec((R, 1), lambda r: (r, 0)),
        ],
        out_specs=[
            pl.BlockSpec((R, 2 * DH), lambda r: (r, 0)),
            pl.BlockSpec((G, 2 * DH), lambda r: (0, 0)),
        ],
        out_shape=[
            jax.ShapeDtypeStruct((N, 2 * DH), jnp.float32),
            jax.ShapeDtypeStruct((G, 2 * DH), jnp.float32),
        ],
    )(fw3, bw3, batch)


def _sage(x3, srcp, dstp, deg, params, zeros128):
    h3 = x3
    n_layers = len(params)
    for i, (wl, wr, b) in enumerate(params):
        cin = h3.shape[0]
        table = h3.reshape(cin * N, 128)
        agg = _agg2 if cin == 2 else _agg4
        s3 = agg(table, srcp, dstp, zeros128).reshape(cin, NPAD, 128)
        wlt3 = wl.T.reshape(cin, 128, DH)
        wrt3 = wr.T.reshape(cin, 128, DH)
        h3 = _dense_layer(deg, s3, h3, wlt3, wrt3, b[None, :],
                          relu=(i < n_layers - 1))
    return h3


def kernel(x, api_feat, root_feat, root_id, fw_edge_idx, bw_edge_idx, batch,
           fw_params, bw_params):
    pad = EPAD - E
    fw_src = fw_edge_idx[0].astype(jnp.int32)
    fw_dst = fw_edge_idx[1].astype(jnp.int32)
    bw_src = bw_edge_idx[0].astype(jnp.int32)
    bw_dst = bw_edge_idx[1].astype(jnp.int32)
    zpad = jnp.zeros((pad,), jnp.int32)
    npad = jnp.full((pad,), N, jnp.int32)
    fw_src_p = jnp.concatenate([fw_src, zpad]).reshape(EPAD // TRB, TRB)
    fw_dst_p = jnp.concatenate([fw_dst, npad]).reshape(EPAD // TRB, TRB)
    bw_src_p = jnp.concatenate([bw_src, zpad]).reshape(EPAD // TRB, TRB)
    bw_dst_p = jnp.concatenate([bw_dst, npad]).reshape(EPAD // TRB, TRB)

    zeros128 = jnp.zeros((NPAD, 128), jnp.float32)
    ones128 = jnp.ones((TRB, 128), jnp.float32)

    dst2 = jnp.concatenate([fw_dst_p, bw_dst_p], axis=0)
    degs = _deg_kernel(dst2, zeros128, ones128).reshape(NC, NPAD, 128)
    deg_fw = degs[0, :N, 0:1]
    deg_bw = degs[1, :N, 0:1]

    x3 = x.reshape(N, DIN // 128, 128).transpose(1, 0, 2)

    fw3 = _sage(x3, fw_src_p, fw_dst_p, deg_fw, fw_params, zeros128)
    bw3 = _sage(x3, bw_src_p, bw_dst_p, deg_bw, bw_params, zeros128)

    x_out, graph_embed = _pool(fw3, bw3, batch.astype(jnp.int32)[:, None])
    return (x_out, graph_embed)
